# Initial kernel scaffold; baseline (speedup 1.0000x reference)
import jax
import jax.numpy as jnp
from jax.experimental import pallas as pl


def kernel(predict, label):
    raise NotImplementedError("write your pallas kernel here")


# speedup vs baseline: 1.7219x; 1.7219x over previous
import functools

import jax
import jax.numpy as jnp
from jax import lax
from jax.experimental import pallas as pl
from jax.experimental.pallas import tpu as pltpu
from jax.experimental.pallas import tpu_sc as plsc

_B = 16384
_C = 1000

_info = plsc.get_sparse_core_info()
_NC = _info.num_cores
_NS = _info.num_subcores
_L = _info.num_lanes
_NW = _NC * _NS
_ROWS_PER_W = _B // _NW
_CHUNK = 128
_NCHUNK = _ROWS_PER_W // _CHUNK

_mesh = plsc.VectorSubcoreMesh(core_axis_name="c", subcore_axis_name="s")


@functools.partial(
    pl.kernel,
    mesh=_mesh,
    out_type=jax.ShapeDtypeStruct((_NW, _L), jnp.float32),
    scratch_types=[
        pltpu.VMEM((_ROWS_PER_W,), jnp.int32),
        pltpu.VMEM((_NCHUNK, _CHUNK), jnp.int32),
        pltpu.VMEM((_ROWS_PER_W,), jnp.float32),
        pltpu.VMEM((_L,), jnp.float32),
        pltpu.SemaphoreType.DMA,
    ],
)
def _gather_partials(pred_hbm, label_hbm, out_hbm, lab_v, idx_v, val_v,
                     acc_v, sem):
    wid = lax.axis_index("s") * _NC + lax.axis_index("c")
    base = wid * _ROWS_PER_W

    pltpu.sync_copy(label_hbm.at[pl.ds(base, _ROWS_PER_W)], lab_v)

    iota = lax.iota(jnp.int32, _L)
    for j in range(_ROWS_PER_W // _L):
        lab = lab_v[pl.ds(j * _L, _L)]
        rows = iota + (base + j * _L)
        idx = rows * _C + lab
        idx_v[j // (_CHUNK // _L), pl.ds((j % (_CHUNK // _L)) * _L, _L)] = idx

    copies = []
    for c in range(_NCHUNK):
        copies.append(
            pltpu.async_copy(
                pred_hbm.at[idx_v.at[c]],
                val_v.at[pl.ds(c * _CHUNK, _CHUNK)],
                sem,
            )
        )
    for cp in copies:
        cp.wait()

    acc = jnp.zeros((_L,), jnp.float32)
    for j in range(_ROWS_PER_W // _L):
        acc = acc + val_v[pl.ds(j * _L, _L)]
    acc_v[...] = acc
    pltpu.sync_copy(acc_v, out_hbm.at[wid])


def kernel(predict, label):
    pred_flat = predict.reshape(-1)
    lab = label.astype(jnp.int32)
    partials = _gather_partials(pred_flat, lab)
    return partials.sum() / predict.shape[0]


# speedup vs baseline: 2.4269x; 1.4095x over previous
import functools

import jax
import jax.numpy as jnp
from jax import lax
from jax.experimental import pallas as pl
from jax.experimental.pallas import tpu as pltpu
from jax.experimental.pallas import tpu_sc as plsc

_B = 16384
_C = 1000

_info = plsc.get_sparse_core_info()
_NC = _info.num_cores
_NS = _info.num_subcores
_L = _info.num_lanes
_NW = _NC * _NS
_ROWS_PER_W = _B // _NW
_CHUNK_ROWS = 32
_NCHUNK = _ROWS_PER_W // _CHUNK_ROWS

_mesh = plsc.VectorSubcoreMesh(core_axis_name="c", subcore_axis_name="s")


@functools.partial(
    pl.kernel,
    mesh=_mesh,
    out_type=jax.ShapeDtypeStruct((_NW, _L), jnp.float32),
    scratch_types=[
        pltpu.VMEM((_ROWS_PER_W,), jnp.int32),
        pltpu.VMEM((2, _CHUNK_ROWS, _C), jnp.float32),
        pltpu.VMEM((_L,), jnp.float32),
        pltpu.SemaphoreType.DMA,
        pltpu.SemaphoreType.DMA,
    ],
)
def _gather_partials(pred_hbm, label_hbm, out_hbm, lab_v, buf_v, acc_v,
                     sem0, sem1):
    wid = lax.axis_index("s") * _NC + lax.axis_index("c")
    base = wid * _ROWS_PER_W

    pltpu.sync_copy(label_hbm.at[pl.ds(base, _ROWS_PER_W)], lab_v)

    sems = (sem0, sem1)

    def start(c, slot):
        return pltpu.async_copy(
            pred_hbm.at[pl.ds(base + c * _CHUNK_ROWS, _CHUNK_ROWS), :],
            buf_v.at[slot],
            sems[slot],
        )

    cp = [start(0, 0), start(1, 1)]
    lane = lax.iota(jnp.int32, _L)
    acc = jnp.zeros((_L,), jnp.float32)
    for c in range(_NCHUNK):
        slot = c % 2
        cp[slot].wait()

        def group(g, a):
            labs = lab_v[pl.ds(c * _CHUNK_ROWS + g * _L, _L)]
            for i in range(_L):
                lab = labs[i]
                w = pl.multiple_of(lab & ~(_L - 1), _L)
                v = buf_v[slot, g * _L + i, pl.ds(w, _L)]
                a = a + jnp.where(lane == (lab & (_L - 1)), v,
                                  jnp.float32(0.0))
            return a

        acc = lax.fori_loop(0, _CHUNK_ROWS // _L, group, acc)
        if c + 2 < _NCHUNK:
            cp[slot] = start(c + 2, slot)

    acc_v[...] = acc
    pltpu.sync_copy(acc_v, out_hbm.at[wid])


def kernel(predict, label):
    lab = label.astype(jnp.int32)
    partials = _gather_partials(predict, lab)
    return partials.sum() / predict.shape[0]


# speedup vs baseline: 5.4116x; 2.2298x over previous
import functools

import jax
import jax.numpy as jnp
from jax import lax
from jax.experimental import pallas as pl
from jax.experimental.pallas import tpu as pltpu
from jax.experimental.pallas import tpu_sc as plsc

_B = 16384
_C = 1000

_info = plsc.get_sparse_core_info()
_NC = _info.num_cores
_NS = _info.num_subcores
_L = _info.num_lanes
_NW = _NC * _NS
_COLS_PER_W = _B // _NW
_CHUNK_COLS = 128
_NCHUNK = _COLS_PER_W // _CHUNK_COLS

_mesh = plsc.VectorSubcoreMesh(core_axis_name="c", subcore_axis_name="s")


@functools.partial(
    pl.kernel,
    mesh=_mesh,
    out_type=jax.ShapeDtypeStruct((_NW, _L), jnp.float32),
    scratch_types=[
        pltpu.VMEM((_COLS_PER_W,), jnp.int32),
        pltpu.VMEM((_C, _CHUNK_COLS), jnp.float32),
        pltpu.VMEM((_L,), jnp.float32),
        pltpu.SemaphoreType.DMA,
    ],
    compiler_params=pltpu.CompilerParams(use_tc_tiling_on_sc=True),
)
def _gather_partials(pred_hbm, label_hbm, out_hbm, lab_v, buf_v, acc_v, sem):
    wid = lax.axis_index("s") * _NC + lax.axis_index("c")
    base = wid * _COLS_PER_W

    pltpu.sync_copy(label_hbm.at[pl.ds(base, _COLS_PER_W)], lab_v)

    lane = lax.iota(jnp.int32, _L)
    acc = jnp.zeros((_L,), jnp.float32)
    for c in range(_NCHUNK):
        pltpu.async_copy(
            pred_hbm.at[:, pl.ds(base + c * _CHUNK_COLS, _CHUNK_COLS)],
            buf_v,
            sem,
        ).wait()

        def group(g, a):
            labs = lab_v[pl.ds(c * _CHUNK_COLS + g * _L, _L)]
            g16 = pl.multiple_of(g * _L, _L)
            for i in range(_L):
                v = buf_v[labs[i], pl.ds(g16, _L)]
                a = a + jnp.where(lane == i, v, jnp.float32(0.0))
            return a

        acc = lax.fori_loop(0, _CHUNK_COLS // _L, group, acc)

    acc_v[...] = acc
    pltpu.sync_copy(acc_v, out_hbm.at[wid])


def kernel(predict, label):
    pred_t = predict.T
    lab = label.astype(jnp.int32)
    partials = _gather_partials(pred_t, lab)
    return partials.sum() / predict.shape[0]


# speedup vs baseline: 6.0745x; 1.1225x over previous
import functools

import jax
import jax.numpy as jnp
from jax import lax
from jax.experimental import pallas as pl
from jax.experimental.pallas import tpu as pltpu
from jax.experimental.pallas import tpu_sc as plsc

_B = 16384
_C = 1000

_info = plsc.get_sparse_core_info()
_NC = _info.num_cores
_NS = _info.num_subcores
_L = _info.num_lanes
_NW = _NC * _NS

_SC_COLS = 8192
_COLS_PER_W = _SC_COLS // _NW
_CHUNK_COLS = 128
_NCHUNK = _COLS_PER_W // _CHUNK_COLS

_TC_BLOCK = 1024
_TC_COLS = _B - _SC_COLS
_TC_BLK0 = _SC_COLS // _TC_BLOCK

_mesh = plsc.VectorSubcoreMesh(core_axis_name="c", subcore_axis_name="s")


@functools.partial(
    pl.kernel,
    mesh=_mesh,
    out_type=jax.ShapeDtypeStruct((_NW, _L), jnp.float32),
    scratch_types=[
        pltpu.VMEM((_COLS_PER_W,), jnp.int32),
        pltpu.VMEM((_C, _CHUNK_COLS), jnp.float32),
        pltpu.VMEM((_L,), jnp.float32),
        pltpu.SemaphoreType.DMA,
    ],
    compiler_params=pltpu.CompilerParams(use_tc_tiling_on_sc=True),
)
def _sc_partials(pred_hbm, label_hbm, out_hbm, lab_v, buf_v, acc_v, sem):
    wid = lax.axis_index("s") * _NC + lax.axis_index("c")
    base = wid * _COLS_PER_W

    pltpu.sync_copy(label_hbm.at[pl.ds(base, _COLS_PER_W)], lab_v)

    lane = lax.iota(jnp.int32, _L)
    acc = jnp.zeros((_L,), jnp.float32)
    for c in range(_NCHUNK):
        pltpu.async_copy(
            pred_hbm.at[:, pl.ds(base + c * _CHUNK_COLS, _CHUNK_COLS)],
            buf_v,
            sem,
        ).wait()

        def group(g, a):
            labs = lab_v[pl.ds(c * _CHUNK_COLS + g * _L, _L)]
            g16 = pl.multiple_of(g * _L, _L)
            for i in range(_L):
                v = buf_v[labs[i], pl.ds(g16, _L)]
                a = a + jnp.where(lane == i, v, jnp.float32(0.0))
            return a

        acc = lax.fori_loop(0, _CHUNK_COLS // _L, group, acc)

    acc_v[...] = acc
    pltpu.sync_copy(acc_v, out_hbm.at[wid])


def _tc_body(pred_ref, lab_ref, out_ref):
    i = pl.program_id(0)

    @pl.when(i == 0)
    def _():
        out_ref[0, 0] = jnp.float32(0.0)

    labs = lab_ref[0, 0, :]
    rows = lax.broadcasted_iota(jnp.int32, (_C, _TC_BLOCK), 0)
    sel = jnp.where(rows == labs[None, :], pred_ref[...], jnp.float32(0.0))
    out_ref[0, 0] += jnp.sum(sel)


_tc_sum = pl.pallas_call(
    _tc_body,
    grid=(_TC_COLS // _TC_BLOCK,),
    in_specs=[
        pl.BlockSpec((_C, _TC_BLOCK), lambda i: (0, _TC_BLK0 + i)),
        pl.BlockSpec((1, 1, _TC_BLOCK), lambda i: (_TC_BLK0 + i, 0, 0)),
    ],
    out_specs=pl.BlockSpec(
        (1, 1), lambda i: (0, 0), memory_space=pltpu.SMEM
    ),
    out_shape=jax.ShapeDtypeStruct((1, 1), jnp.float32),
    compiler_params=pltpu.CompilerParams(
        dimension_semantics=("arbitrary",)
    ),
)


def kernel(predict, label):
    pred_t = predict.T
    lab = label.astype(jnp.int32)
    sc_part = _sc_partials(pred_t, lab)
    tc_part = _tc_sum(pred_t, lab.reshape(_B // _TC_BLOCK, 1, _TC_BLOCK))
    return (sc_part.sum() + tc_part[0, 0]) / predict.shape[0]
